# baseline (speedup 1.0000x reference)
import functools

import jax
import jax.numpy as jnp
from jax import lax
from jax.experimental import pallas as pl
from jax.experimental.pallas import tpu as pltpu
from jax.experimental.pallas import tpu_sc as plsc

N = 10000
E = 320000
D = 128
H = 128
DE = 16

CHUNK = 128
NCHUNKS = E // CHUNK
NC = 2
NS = 16
NW = NC * NS
L = 16
NP = 10240
ROWS_PER_TILE = NP // NS

S = 5
ES = E // S
NCH_S = NCHUNKS // S

_mesh = plsc.VectorSubcoreMesh(core_axis_name="c", subcore_axis_name="s")
_sc_params = pltpu.CompilerParams(needs_layout_passes=False)


def _make_gather(nch):
    n_edges = nch * CHUNK
    iters = (nch + NW - 1) // NW
    iters += iters % 2
    half = iters // 2

    @functools.partial(
        pl.kernel,
        out_type=(
            jax.ShapeDtypeStruct((n_edges, D), jnp.float32),
            jax.ShapeDtypeStruct((n_edges, D), jnp.float32),
            jax.ShapeDtypeStruct((n_edges, 4), jnp.float32),
        ),
        mesh=_mesh,
        compiler_params=_sc_params,
        scratch_types=[
            pltpu.VMEM((3 * N,), jnp.float32),
            pltpu.VMEM((CHUNK,), jnp.int32),
            pltpu.VMEM((CHUNK,), jnp.int32),
            pltpu.VMEM((CHUNK,), jnp.int32),
            pltpu.VMEM((CHUNK,), jnp.int32),
            pltpu.VMEM((CHUNK, D), jnp.float32),
            pltpu.VMEM((CHUNK, D), jnp.float32),
            pltpu.VMEM((CHUNK, D), jnp.float32),
            pltpu.VMEM((CHUNK, D), jnp.float32),
            pltpu.VMEM((CHUNK, 4), jnp.float32),
            pltpu.VMEM((CHUNK, 4), jnp.float32),
            pltpu.SemaphoreType.DMA,
            pltpu.SemaphoreType.DMA,
            pltpu.SemaphoreType.DMA,
            pltpu.SemaphoreType.DMA,
            pltpu.SemaphoreType.DMA,
            pltpu.SemaphoreType.DMA,
        ],
    )
    def gather(table_hbm, ctab_hbm, starts_hbm, ends_hbm,
               gs_hbm, ge_hbm, cdn_hbm,
               ctab_v, idx_s0, idx_e0, idx_s1, idx_e1,
               rows_s0, rows_e0, rows_s1, rows_e1, cdn0, cdn1,
               sem_i0, sem_i1, sem_g0, sem_g1, sem_w0, sem_w1):
        wid = lax.axis_index("s") * NC + lax.axis_index("c")
        pltpu.sync_copy(ctab_hbm, ctab_v)

        bufs = (
            (idx_s0, idx_e0, rows_s0, rows_e0, cdn0, sem_i0, sem_g0, sem_w0),
            (idx_s1, idx_e1, rows_s1, rows_e1, cdn1, sem_i1, sem_g1, sem_w1),
        )

        def start_idx(p, chunk):
            idx_s, idx_e, _, _, _, sem_i, _, _ = bufs[p]
            pltpu.async_copy(starts_hbm.at[chunk], idx_s, sem_i)
            pltpu.async_copy(ends_hbm.at[chunk], idx_e, sem_i)

        def wait_idx(p, chunk):
            idx_s, idx_e, _, _, _, sem_i, _, _ = bufs[p]
            pltpu.make_async_copy(starts_hbm.at[chunk], idx_s, sem_i).wait()
            pltpu.make_async_copy(ends_hbm.at[chunk], idx_e, sem_i).wait()

        def start_gather(p):
            idx_s, idx_e, rows_s, rows_e, _, _, sem_g, _ = bufs[p]
            pltpu.async_copy(table_hbm.at[idx_s], rows_s, sem_g)
            pltpu.async_copy(table_hbm.at[idx_e], rows_e, sem_g)

        def wait_gather(p):
            idx_s, idx_e, rows_s, rows_e, _, _, sem_g, _ = bufs[p]
            pltpu.make_async_copy(table_hbm.at[idx_s], rows_s, sem_g).wait()
            pltpu.make_async_copy(table_hbm.at[idx_e], rows_e, sem_g).wait()

        def start_write(p, chunk):
            _, _, rows_s, rows_e, cdn_v, _, _, sem_w = bufs[p]
            sl = pl.ds(chunk * CHUNK, CHUNK)
            pltpu.async_copy(rows_s, gs_hbm.at[sl], sem_w)
            pltpu.async_copy(rows_e, ge_hbm.at[sl], sem_w)
            pltpu.async_copy(cdn_v, cdn_hbm.at[sl], sem_w)

        def wait_write(p, chunk):
            _, _, rows_s, rows_e, cdn_v, _, _, sem_w = bufs[p]
            sl = pl.ds(chunk * CHUNK, CHUNK)
            pltpu.make_async_copy(rows_s, gs_hbm.at[sl], sem_w).wait()
            pltpu.make_async_copy(rows_e, ge_hbm.at[sl], sem_w).wait()
            pltpu.make_async_copy(cdn_v, cdn_hbm.at[sl], sem_w).wait()

        def coords(p):
            idx_s, idx_e, _, _, cdn_v, _, _, _ = bufs[p]
            for j in range(CHUNK // L):
                lanes = lax.iota(jnp.int32, L) + j * L
                i_s = idx_s[pl.ds(j * L, L)] * 3
                i_e = idx_e[pl.ds(j * L, L)] * 3
                cdx = plsc.load_gather(ctab_v, [i_s]) - plsc.load_gather(ctab_v, [i_e])
                cdy = plsc.load_gather(ctab_v, [i_s + 1]) - plsc.load_gather(ctab_v, [i_e + 1])
                cdz = plsc.load_gather(ctab_v, [i_s + 2]) - plsc.load_gather(ctab_v, [i_e + 2])
                n2 = cdx * cdx + cdy * cdy + cdz * cdz + 1e-12
                c0 = jnp.zeros((L,), jnp.int32)
                plsc.store_scatter(cdn_v, [lanes, c0], cdx)
                plsc.store_scatter(cdn_v, [lanes, c0 + 1], cdy)
                plsc.store_scatter(cdn_v, [lanes, c0 + 2], cdz)
                plsc.store_scatter(cdn_v, [lanes, c0 + 3], n2)

        start_idx(0, wid)

        def body(k, carry):
            for p in range(2):
                i = 2 * k + p
                c_i = wid + i * NW
                c_n = c_i + NW
                c_o = c_i - 2 * NW

                @pl.when(c_i < nch)
                def _():
                    wait_idx(p, c_i)

                @pl.when(jnp.logical_and(i >= 2, c_o < nch))
                def _():
                    wait_write(p, c_o)

                @pl.when(c_i < nch)
                def _():
                    start_gather(p)

                @pl.when(c_n < nch)
                def _():
                    start_idx(1 - p, c_n)

                @pl.when(c_i < nch)
                def _():
                    coords(p)
                    wait_gather(p)
                    start_write(p, c_i)

            return carry

        lax.fori_loop(0, half, body, 0)

        for p in range(2):
            i = iters - 2 + p
            c_i = wid + i * NW

            @pl.when(c_i < nch)
            def _():
                wait_write(p, c_i)

    return gather


def _make_scatter_m(nch):
    chunks_per_core = nch // NC
    iters = (chunks_per_core + NS - 1) // NS
    iters += iters % 2

    @functools.partial(
        pl.kernel,
        out_type=jax.ShapeDtypeStruct((NC, NP, D), jnp.float32),
        mesh=_mesh,
        compiler_params=_sc_params,
        scratch_types=[
            pltpu.VMEM_SHARED((NP, D), jnp.float32),
            pltpu.VMEM((CHUNK,), jnp.int32),
            pltpu.VMEM((CHUNK,), jnp.int32),
            pltpu.VMEM((CHUNK, D), jnp.float32),
            pltpu.VMEM((CHUNK, D), jnp.float32),
            pltpu.SemaphoreType.DMA,
            pltpu.SemaphoreType.DMA,
            pltpu.SemaphoreType.DMA,
            pltpu.SemaphoreType.DMA,
        ],
    )
    def scatter_m(mvals_hbm, starts_hbm, zeros_hbm, out_m_hbm, acc_m,
                  idx0, idx1, rows0, rows1, sem_l0, sem_l1, sem_a0, sem_a1):
        cid = lax.axis_index("c")
        sid = lax.axis_index("s")
        pltpu.sync_copy(zeros_hbm,
                        acc_m.at[pl.ds(sid * ROWS_PER_TILE, ROWS_PER_TILE)])
        plsc.subcore_barrier()

        bufs = ((idx0, rows0, sem_l0, sem_a0), (idx1, rows1, sem_l1, sem_a1))

        def chunk_of(local):
            return cid * chunks_per_core + local

        def start_loads(p, local):
            idx_v, rows_v, sem_l, _ = bufs[p]
            c = chunk_of(local)
            pltpu.async_copy(starts_hbm.at[c], idx_v, sem_l)
            pltpu.async_copy(mvals_hbm.at[pl.ds(c * CHUNK, CHUNK)], rows_v, sem_l)

        def wait_loads(p, local):
            idx_v, rows_v, sem_l, _ = bufs[p]
            c = chunk_of(local)
            pltpu.make_async_copy(starts_hbm.at[c], idx_v, sem_l).wait()
            pltpu.make_async_copy(mvals_hbm.at[pl.ds(c * CHUNK, CHUNK)], rows_v, sem_l).wait()

        def start_add(p):
            idx_v, rows_v, _, sem_a = bufs[p]
            pltpu.async_copy(rows_v, acc_m.at[idx_v], sem_a, add=True)

        def wait_add(p):
            idx_v, rows_v, _, sem_a = bufs[p]
            pltpu.make_async_copy(rows_v, acc_m.at[idx_v], sem_a).wait()

        start_loads(0, sid)

        def body(k, carry):
            for p in range(2):
                i = 2 * k + p
                l_i = sid + i * NS
                l_n = l_i + NS
                l_o = l_i - 2 * NS

                @pl.when(jnp.logical_and(i >= 2, l_o < chunks_per_core))
                def _():
                    wait_add(p)

                @pl.when(l_i < chunks_per_core)
                def _():
                    wait_loads(p, l_i)
                    start_add(p)

                @pl.when(l_n < chunks_per_core)
                def _():
                    start_loads(1 - p, l_n)

            return carry

        lax.fori_loop(0, iters // 2, body, 0)

        for p in range(2):
            i = iters - 2 + p
            l_i = sid + i * NS

            @pl.when(l_i < chunks_per_core)
            def _():
                wait_add(p)

        plsc.subcore_barrier()
        pltpu.sync_copy(
            acc_m.at[pl.ds(sid * ROWS_PER_TILE, ROWS_PER_TILE)],
            out_m_hbm.at[cid, pl.ds(sid * ROWS_PER_TILE, ROWS_PER_TILE)],
        )

    return scatter_m


def _make_scatter_c(nch):
    iters = (nch + NW - 1) // NW
    iters += iters % 2

    @functools.partial(
        pl.kernel,
        out_type=jax.ShapeDtypeStruct((NW, 3 * N), jnp.float32),
        mesh=_mesh,
        compiler_params=_sc_params,
        scratch_types=[
            pltpu.VMEM((3 * N,), jnp.float32),
            pltpu.VMEM((CHUNK,), jnp.int32),
            pltpu.VMEM((CHUNK,), jnp.int32),
            pltpu.VMEM((CHUNK, 4), jnp.float32),
            pltpu.VMEM((CHUNK, 4), jnp.float32),
            pltpu.SemaphoreType.DMA,
            pltpu.SemaphoreType.DMA,
        ],
    )
    def scatter_c(cterm_hbm, starts_hbm, zeros_c_hbm, out_c_hbm, acc_c,
                  idx0, idx1, rows0, rows1, sem_l0, sem_l1):
        cid = lax.axis_index("c")
        sid = lax.axis_index("s")
        wid = sid * NC + cid
        pltpu.sync_copy(zeros_c_hbm, acc_c)

        bufs = ((idx0, rows0, sem_l0), (idx1, rows1, sem_l1))

        def start_loads(p, chunk):
            idx_v, rows_c, sem_l = bufs[p]
            pltpu.async_copy(starts_hbm.at[chunk], idx_v, sem_l)
            pltpu.async_copy(cterm_hbm.at[pl.ds(chunk * CHUNK, CHUNK)], rows_c, sem_l)

        def wait_loads(p, chunk):
            idx_v, rows_c, sem_l = bufs[p]
            pltpu.make_async_copy(starts_hbm.at[chunk], idx_v, sem_l).wait()
            pltpu.make_async_copy(cterm_hbm.at[pl.ds(chunk * CHUNK, CHUNK)], rows_c, sem_l).wait()

        def compute(p):
            idx_v, rows_c, _ = bufs[p]
            for j in range(CHUNK // L):
                lanes = lax.iota(jnp.int32, L) + j * L
                tgt = idx_v[pl.ds(j * L, L)] * 3
                c0 = jnp.zeros((L,), jnp.int32)
                cx = plsc.load_gather(rows_c, [lanes, c0])
                cy = plsc.load_gather(rows_c, [lanes, c0 + 1])
                cz = plsc.load_gather(rows_c, [lanes, c0 + 2])
                plsc.addupdate_scatter(acc_c, [tgt], cx)
                plsc.addupdate_scatter(acc_c, [tgt + 1], cy)
                plsc.addupdate_scatter(acc_c, [tgt + 2], cz)

        start_loads(0, wid)

        def body(k, carry):
            for p in range(2):
                i = 2 * k + p
                c_i = wid + i * NW
                c_n = c_i + NW

                @pl.when(c_n < nch)
                def _():
                    start_loads(1 - p, c_n)

                @pl.when(c_i < nch)
                def _():
                    wait_loads(p, c_i)
                    compute(p)

            return carry

        lax.fori_loop(0, iters // 2, body, 0)
        pltpu.sync_copy(acc_c, out_c_hbm.at[wid])

    return scatter_c


_gather_s = _make_gather(NCH_S)
_scatter_m_s = _make_scatter_m(NCH_S)
_scatter_c_s = _make_scatter_c(NCH_S)


def _edge_body(gs, ge, cdn, ef, wa, wb, wcr, wd, be1, we2, be2, wc1, bc1,
               wc2, bc2, wi, bi, out_m, out_c):
    f32 = jnp.float32
    cd = cdn[:, 0:3]
    norm = jnp.sqrt(cdn[:, 3:4])
    x = (jnp.dot(gs[...], wa[...], preferred_element_type=f32)
         + jnp.dot(ge[...], wb[...], preferred_element_type=f32)
         + jnp.dot(ef[...], wd[...], preferred_element_type=f32)
         + norm * wcr[...] + be1[...])
    m = x * jax.nn.sigmoid(x)
    x = jnp.dot(m, we2[...], preferred_element_type=f32) + be2[...]
    m = x * jax.nn.sigmoid(x)
    x = jnp.dot(m, wc1[...], preferred_element_type=f32) + bc1[...]
    c = x * jax.nn.sigmoid(x)
    c = jnp.dot(c, wc2[...], preferred_element_type=f32) + bc2[...]
    gate = jax.nn.sigmoid(jnp.dot(m, wi[...], preferred_element_type=f32) + bi[...])
    out_m[...] = m * gate
    bsz = cd.shape[0]
    out_c[...] = jnp.concatenate([cd * c, jnp.zeros((bsz, 1), f32)], axis=1)


def _pc_reduce_body(*refs):
    out = refs[-1]
    acc = refs[0][...]
    for r in refs[1:-1]:
        acc = acc + r[...]
    out[...] = jnp.sum(acc, axis=0)


def _node_body(nf, aggc3, wv1, bv1, wv2, bv2, wn1a, wn1b, bn1, wn2, bn2,
               *pms_and_out):
    pms = pms_and_out[:-1]
    out = pms_and_out[-1]
    f32 = jnp.float32
    coords = nf[:, 0:3]
    vels = nf[:, 3:6]
    h = nf[:, 6:6 + D]
    aggm = pms[0][0] + pms[0][1]
    for pm in pms[1:]:
        aggm = aggm + pm[0] + pm[1]
    aggc = aggc3[...]
    x = jnp.dot(h, wv1[...], preferred_element_type=f32) + bv1[...]
    v = x * jax.nn.sigmoid(x)
    v = jnp.dot(v, wv2[...], preferred_element_type=f32) + bv2[...]
    coords_new = coords + aggc + v * vels
    x = (jnp.dot(h, wn1a[...], preferred_element_type=f32)
         + jnp.dot(aggm, wn1b[...], preferred_element_type=f32) + bn1[...])
    hn = x * jax.nn.sigmoid(x)
    hn = jnp.dot(hn, wn2[...], preferred_element_type=f32) + bn2[...]
    out[...] = jnp.concatenate([coords_new, vels, h + hn], axis=1)


def kernel(node_features_input, edge_features, W_e1, b_e1, W_e2, b_e2,
           W_c1, b_c1, W_c2, b_c2, W_v1, b_v1, W_v2, b_v2, W_n1, b_n1,
           W_n2, b_n2, W_i, b_i, edge_indices):
    f32 = jnp.float32
    nf = node_features_input
    h_table = nf[:, 6:]
    ctab = nf[:, :3].reshape(-1)
    starts2d = edge_indices[0].reshape(NCHUNKS, CHUNK)
    ends2d = edge_indices[1].reshape(NCHUNKS, CHUNK)

    zeros_m = jnp.zeros((ROWS_PER_TILE, D), f32)
    zeros_c = jnp.zeros((3 * N,), f32)

    BE = 4000
    grid_e = ES // BE
    full = lambda i: (0, 0)
    edge_weight_specs = [
        pl.BlockSpec((D, H), full),
        pl.BlockSpec((D, H), full),
        pl.BlockSpec((1, H), full),
        pl.BlockSpec((DE, H), full),
        pl.BlockSpec((1, H), full),
        pl.BlockSpec((H, H), full),
        pl.BlockSpec((1, H), full),
        pl.BlockSpec((H, H), full),
        pl.BlockSpec((1, H), full),
        pl.BlockSpec((H, 1), full),
        pl.BlockSpec((1, 1), full),
        pl.BlockSpec((H, 1), full),
        pl.BlockSpec((1, 1), full),
    ]
    edge_weights = (
        W_e1[:D], W_e1[D:2 * D], W_e1[2 * D:2 * D + 1], W_e1[2 * D + 1:],
        b_e1.reshape(1, H), W_e2, b_e2.reshape(1, H), W_c1,
        b_c1.reshape(1, H), W_c2, b_c2.reshape(1, 1), W_i, b_i.reshape(1, 1))

    partial_ms = []
    partial_cs = []
    for k in range(S):
        st_k = lax.slice_in_dim(starts2d, k * NCH_S, (k + 1) * NCH_S)
        en_k = lax.slice_in_dim(ends2d, k * NCH_S, (k + 1) * NCH_S)
        gs, ge, cdn = _gather_s(h_table, ctab, st_k, en_k)

        row_block = lambda i: (i, 0)
        ef_block = lambda i, k=k: (k * grid_e + i, 0)
        mvals, cterm = pl.pallas_call(
            _edge_body,
            grid=(grid_e,),
            in_specs=[
                pl.BlockSpec((BE, D), row_block),
                pl.BlockSpec((BE, D), row_block),
                pl.BlockSpec((BE, 4), row_block),
                pl.BlockSpec((BE, DE), ef_block),
            ] + edge_weight_specs,
            out_specs=[
                pl.BlockSpec((BE, D), row_block),
                pl.BlockSpec((BE, 4), row_block),
            ],
            out_shape=[
                jax.ShapeDtypeStruct((ES, D), f32),
                jax.ShapeDtypeStruct((ES, 4), f32),
            ],
        )(gs, ge, cdn, edge_features, *edge_weights)

        partial_cs.append(_scatter_c_s(cterm, st_k, zeros_c))
        partial_ms.append(_scatter_m_s(mvals, st_k, zeros_m))

    agg_c = pl.pallas_call(
        _pc_reduce_body,
        grid=(1,),
        in_specs=[pl.BlockSpec((NW, 3 * N), lambda i: (0, 0))] * S,
        out_specs=pl.BlockSpec((3 * N,), lambda i: (0,)),
        out_shape=jax.ShapeDtypeStruct((3 * N,), f32),
    )(*partial_cs).reshape(N, 3)

    BN = 1000
    grid_n = N // BN
    out = pl.pallas_call(
        _node_body,
        grid=(grid_n,),
        in_specs=[
            pl.BlockSpec((BN, 6 + D), lambda i: (i, 0)),
            pl.BlockSpec((BN, 3), lambda i: (i, 0)),
            pl.BlockSpec((D, H), lambda i: (0, 0)),
            pl.BlockSpec((1, H), lambda i: (0, 0)),
            pl.BlockSpec((H, 1), lambda i: (0, 0)),
            pl.BlockSpec((1, 1), lambda i: (0, 0)),
            pl.BlockSpec((D, H), lambda i: (0, 0)),
            pl.BlockSpec((H, H), lambda i: (0, 0)),
            pl.BlockSpec((1, H), lambda i: (0, 0)),
            pl.BlockSpec((H, D), lambda i: (0, 0)),
            pl.BlockSpec((1, D), lambda i: (0, 0)),
        ] + [pl.BlockSpec((NC, BN, D), lambda i: (0, i, 0))] * S,
        out_specs=pl.BlockSpec((BN, 6 + D), lambda i: (i, 0)),
        out_shape=jax.ShapeDtypeStruct((N, 6 + D), f32),
    )(nf, agg_c, W_v1, b_v1.reshape(1, H), W_v2, b_v2.reshape(1, 1),
      W_n1[:D], W_n1[D:], b_n1.reshape(1, H), W_n2, b_n2.reshape(1, D),
      *partial_ms)

    return out


# speedup vs baseline: 1.0860x; 1.0860x over previous
import functools

import jax
import jax.numpy as jnp
from jax import lax
from jax.experimental import pallas as pl
from jax.experimental.pallas import tpu as pltpu
from jax.experimental.pallas import tpu_sc as plsc

N = 10000
E = 320000
D = 128
H = 128
DE = 16

CHUNK = 128
NCHUNKS = E // CHUNK
NC = 2
NS = 16
NW = NC * NS
L = 16
NP = 10240
ROWS_PER_TILE = NP // NS

S = 2
ES = E // S
NCH_S = NCHUNKS // S

_mesh = plsc.VectorSubcoreMesh(core_axis_name="c", subcore_axis_name="s")
_sc_params = pltpu.CompilerParams(needs_layout_passes=False)


def _make_gather(nch):
    n_edges = nch * CHUNK
    iters = (nch + NW - 1) // NW
    iters += iters % 2
    half = iters // 2

    @functools.partial(
        pl.kernel,
        out_type=(
            jax.ShapeDtypeStruct((n_edges, D), jnp.float32),
            jax.ShapeDtypeStruct((n_edges, D), jnp.float32),
            jax.ShapeDtypeStruct((n_edges, 4), jnp.float32),
        ),
        mesh=_mesh,
        compiler_params=_sc_params,
        scratch_types=[
            pltpu.VMEM((3 * N,), jnp.float32),
            pltpu.VMEM((CHUNK,), jnp.int32),
            pltpu.VMEM((CHUNK,), jnp.int32),
            pltpu.VMEM((CHUNK,), jnp.int32),
            pltpu.VMEM((CHUNK,), jnp.int32),
            pltpu.VMEM((CHUNK, D), jnp.float32),
            pltpu.VMEM((CHUNK, D), jnp.float32),
            pltpu.VMEM((CHUNK, D), jnp.float32),
            pltpu.VMEM((CHUNK, D), jnp.float32),
            pltpu.VMEM((CHUNK, 4), jnp.float32),
            pltpu.VMEM((CHUNK, 4), jnp.float32),
            pltpu.SemaphoreType.DMA,
            pltpu.SemaphoreType.DMA,
            pltpu.SemaphoreType.DMA,
            pltpu.SemaphoreType.DMA,
            pltpu.SemaphoreType.DMA,
            pltpu.SemaphoreType.DMA,
        ],
    )
    def gather(table_hbm, ctab_hbm, starts_hbm, ends_hbm,
               gs_hbm, ge_hbm, cdn_hbm,
               ctab_v, idx_s0, idx_e0, idx_s1, idx_e1,
               rows_s0, rows_e0, rows_s1, rows_e1, cdn0, cdn1,
               sem_i0, sem_i1, sem_g0, sem_g1, sem_w0, sem_w1):
        wid = lax.axis_index("s") * NC + lax.axis_index("c")
        pltpu.sync_copy(ctab_hbm, ctab_v)

        bufs = (
            (idx_s0, idx_e0, rows_s0, rows_e0, cdn0, sem_i0, sem_g0, sem_w0),
            (idx_s1, idx_e1, rows_s1, rows_e1, cdn1, sem_i1, sem_g1, sem_w1),
        )

        def start_idx(p, chunk):
            idx_s, idx_e, _, _, _, sem_i, _, _ = bufs[p]
            pltpu.async_copy(starts_hbm.at[chunk], idx_s, sem_i)
            pltpu.async_copy(ends_hbm.at[chunk], idx_e, sem_i)

        def wait_idx(p, chunk):
            idx_s, idx_e, _, _, _, sem_i, _, _ = bufs[p]
            pltpu.make_async_copy(starts_hbm.at[chunk], idx_s, sem_i).wait()
            pltpu.make_async_copy(ends_hbm.at[chunk], idx_e, sem_i).wait()

        def start_gather(p):
            idx_s, idx_e, rows_s, rows_e, _, _, sem_g, _ = bufs[p]
            pltpu.async_copy(table_hbm.at[idx_s], rows_s, sem_g)
            pltpu.async_copy(table_hbm.at[idx_e], rows_e, sem_g)

        def wait_gather(p):
            idx_s, idx_e, rows_s, rows_e, _, _, sem_g, _ = bufs[p]
            pltpu.make_async_copy(table_hbm.at[idx_s], rows_s, sem_g).wait()
            pltpu.make_async_copy(table_hbm.at[idx_e], rows_e, sem_g).wait()

        def start_write(p, chunk):
            _, _, rows_s, rows_e, cdn_v, _, _, sem_w = bufs[p]
            sl = pl.ds(chunk * CHUNK, CHUNK)
            pltpu.async_copy(rows_s, gs_hbm.at[sl], sem_w)
            pltpu.async_copy(rows_e, ge_hbm.at[sl], sem_w)
            pltpu.async_copy(cdn_v, cdn_hbm.at[sl], sem_w)

        def wait_write(p, chunk):
            _, _, rows_s, rows_e, cdn_v, _, _, sem_w = bufs[p]
            sl = pl.ds(chunk * CHUNK, CHUNK)
            pltpu.make_async_copy(rows_s, gs_hbm.at[sl], sem_w).wait()
            pltpu.make_async_copy(rows_e, ge_hbm.at[sl], sem_w).wait()
            pltpu.make_async_copy(cdn_v, cdn_hbm.at[sl], sem_w).wait()

        def coords(p):
            idx_s, idx_e, _, _, cdn_v, _, _, _ = bufs[p]
            for j in range(CHUNK // L):
                lanes = lax.iota(jnp.int32, L) + j * L
                i_s = idx_s[pl.ds(j * L, L)] * 3
                i_e = idx_e[pl.ds(j * L, L)] * 3
                cdx = plsc.load_gather(ctab_v, [i_s]) - plsc.load_gather(ctab_v, [i_e])
                cdy = plsc.load_gather(ctab_v, [i_s + 1]) - plsc.load_gather(ctab_v, [i_e + 1])
                cdz = plsc.load_gather(ctab_v, [i_s + 2]) - plsc.load_gather(ctab_v, [i_e + 2])
                n2 = cdx * cdx + cdy * cdy + cdz * cdz + 1e-12
                c0 = jnp.zeros((L,), jnp.int32)
                plsc.store_scatter(cdn_v, [lanes, c0], cdx)
                plsc.store_scatter(cdn_v, [lanes, c0 + 1], cdy)
                plsc.store_scatter(cdn_v, [lanes, c0 + 2], cdz)
                plsc.store_scatter(cdn_v, [lanes, c0 + 3], n2)

        start_idx(0, wid)

        def body(k, carry):
            for p in range(2):
                i = 2 * k + p
                c_i = wid + i * NW
                c_n = c_i + NW
                c_o = c_i - 2 * NW

                @pl.when(c_i < nch)
                def _():
                    wait_idx(p, c_i)

                @pl.when(jnp.logical_and(i >= 2, c_o < nch))
                def _():
                    wait_write(p, c_o)

                @pl.when(c_i < nch)
                def _():
                    start_gather(p)

                @pl.when(c_n < nch)
                def _():
                    start_idx(1 - p, c_n)

                @pl.when(c_i < nch)
                def _():
                    coords(p)
                    wait_gather(p)
                    start_write(p, c_i)

            return carry

        lax.fori_loop(0, half, body, 0)

        for p in range(2):
            i = iters - 2 + p
            c_i = wid + i * NW

            @pl.when(c_i < nch)
            def _():
                wait_write(p, c_i)

    return gather


def _make_scatter_m(nch):
    chunks_per_core = nch // NC
    iters = (chunks_per_core + NS - 1) // NS
    iters += iters % 2

    @functools.partial(
        pl.kernel,
        out_type=jax.ShapeDtypeStruct((NC, NP, D), jnp.float32),
        mesh=_mesh,
        compiler_params=_sc_params,
        scratch_types=[
            pltpu.VMEM_SHARED((NP, D), jnp.float32),
            pltpu.VMEM((CHUNK,), jnp.int32),
            pltpu.VMEM((CHUNK,), jnp.int32),
            pltpu.VMEM((CHUNK, D), jnp.float32),
            pltpu.VMEM((CHUNK, D), jnp.float32),
            pltpu.SemaphoreType.DMA,
            pltpu.SemaphoreType.DMA,
            pltpu.SemaphoreType.DMA,
            pltpu.SemaphoreType.DMA,
        ],
    )
    def scatter_m(mvals_hbm, starts_hbm, zeros_hbm, out_m_hbm, acc_m,
                  idx0, idx1, rows0, rows1, sem_l0, sem_l1, sem_a0, sem_a1):
        cid = lax.axis_index("c")
        sid = lax.axis_index("s")
        pltpu.sync_copy(zeros_hbm,
                        acc_m.at[pl.ds(sid * ROWS_PER_TILE, ROWS_PER_TILE)])
        plsc.subcore_barrier()

        bufs = ((idx0, rows0, sem_l0, sem_a0), (idx1, rows1, sem_l1, sem_a1))

        def chunk_of(local):
            return cid * chunks_per_core + local

        def start_loads(p, local):
            idx_v, rows_v, sem_l, _ = bufs[p]
            c = chunk_of(local)
            pltpu.async_copy(starts_hbm.at[c], idx_v, sem_l)
            pltpu.async_copy(mvals_hbm.at[pl.ds(c * CHUNK, CHUNK)], rows_v, sem_l)

        def wait_loads(p, local):
            idx_v, rows_v, sem_l, _ = bufs[p]
            c = chunk_of(local)
            pltpu.make_async_copy(starts_hbm.at[c], idx_v, sem_l).wait()
            pltpu.make_async_copy(mvals_hbm.at[pl.ds(c * CHUNK, CHUNK)], rows_v, sem_l).wait()

        def start_add(p):
            idx_v, rows_v, _, sem_a = bufs[p]
            pltpu.async_copy(rows_v, acc_m.at[idx_v], sem_a, add=True)

        def wait_add(p):
            idx_v, rows_v, _, sem_a = bufs[p]
            pltpu.make_async_copy(rows_v, acc_m.at[idx_v], sem_a).wait()

        start_loads(0, sid)

        def body(k, carry):
            for p in range(2):
                i = 2 * k + p
                l_i = sid + i * NS
                l_n = l_i + NS
                l_o = l_i - 2 * NS

                @pl.when(jnp.logical_and(i >= 2, l_o < chunks_per_core))
                def _():
                    wait_add(p)

                @pl.when(l_i < chunks_per_core)
                def _():
                    wait_loads(p, l_i)
                    start_add(p)

                @pl.when(l_n < chunks_per_core)
                def _():
                    start_loads(1 - p, l_n)

            return carry

        lax.fori_loop(0, iters // 2, body, 0)

        for p in range(2):
            i = iters - 2 + p
            l_i = sid + i * NS

            @pl.when(l_i < chunks_per_core)
            def _():
                wait_add(p)

        plsc.subcore_barrier()
        pltpu.sync_copy(
            acc_m.at[pl.ds(sid * ROWS_PER_TILE, ROWS_PER_TILE)],
            out_m_hbm.at[cid, pl.ds(sid * ROWS_PER_TILE, ROWS_PER_TILE)],
        )

    return scatter_m


def _make_scatter_c(nch):
    iters = (nch + NW - 1) // NW
    iters += iters % 2

    @functools.partial(
        pl.kernel,
        out_type=jax.ShapeDtypeStruct((NW, 3 * N), jnp.float32),
        mesh=_mesh,
        compiler_params=_sc_params,
        scratch_types=[
            pltpu.VMEM((3 * N,), jnp.float32),
            pltpu.VMEM((CHUNK,), jnp.int32),
            pltpu.VMEM((CHUNK,), jnp.int32),
            pltpu.VMEM((CHUNK, 4), jnp.float32),
            pltpu.VMEM((CHUNK, 4), jnp.float32),
            pltpu.SemaphoreType.DMA,
            pltpu.SemaphoreType.DMA,
        ],
    )
    def scatter_c(cterm_hbm, starts_hbm, zeros_c_hbm, out_c_hbm, acc_c,
                  idx0, idx1, rows0, rows1, sem_l0, sem_l1):
        cid = lax.axis_index("c")
        sid = lax.axis_index("s")
        wid = sid * NC + cid
        pltpu.sync_copy(zeros_c_hbm, acc_c)

        bufs = ((idx0, rows0, sem_l0), (idx1, rows1, sem_l1))

        def start_loads(p, chunk):
            idx_v, rows_c, sem_l = bufs[p]
            pltpu.async_copy(starts_hbm.at[chunk], idx_v, sem_l)
            pltpu.async_copy(cterm_hbm.at[pl.ds(chunk * CHUNK, CHUNK)], rows_c, sem_l)

        def wait_loads(p, chunk):
            idx_v, rows_c, sem_l = bufs[p]
            pltpu.make_async_copy(starts_hbm.at[chunk], idx_v, sem_l).wait()
            pltpu.make_async_copy(cterm_hbm.at[pl.ds(chunk * CHUNK, CHUNK)], rows_c, sem_l).wait()

        def compute(p):
            idx_v, rows_c, _ = bufs[p]
            for j in range(CHUNK // L):
                lanes = lax.iota(jnp.int32, L) + j * L
                tgt = idx_v[pl.ds(j * L, L)] * 3
                c0 = jnp.zeros((L,), jnp.int32)
                cx = plsc.load_gather(rows_c, [lanes, c0])
                cy = plsc.load_gather(rows_c, [lanes, c0 + 1])
                cz = plsc.load_gather(rows_c, [lanes, c0 + 2])
                plsc.addupdate_scatter(acc_c, [tgt], cx)
                plsc.addupdate_scatter(acc_c, [tgt + 1], cy)
                plsc.addupdate_scatter(acc_c, [tgt + 2], cz)

        start_loads(0, wid)

        def body(k, carry):
            for p in range(2):
                i = 2 * k + p
                c_i = wid + i * NW
                c_n = c_i + NW

                @pl.when(c_n < nch)
                def _():
                    start_loads(1 - p, c_n)

                @pl.when(c_i < nch)
                def _():
                    wait_loads(p, c_i)
                    compute(p)

            return carry

        lax.fori_loop(0, iters // 2, body, 0)
        pltpu.sync_copy(acc_c, out_c_hbm.at[wid])

    return scatter_c


_gather_s = _make_gather(NCH_S)
_scatter_m_s = _make_scatter_m(NCH_S)
_scatter_c_s = _make_scatter_c(NCH_S)


def _edge_body(gs, ge, cdn, ef, wa, wb, wcr, wd, be1, we2, be2, wc1, bc1,
               wc2, bc2, wi, bi, out_m, out_c):
    f32 = jnp.float32
    cd = cdn[:, 0:3]
    norm = jnp.sqrt(cdn[:, 3:4])
    x = (jnp.dot(gs[...], wa[...], preferred_element_type=f32)
         + jnp.dot(ge[...], wb[...], preferred_element_type=f32)
         + jnp.dot(ef[...], wd[...], preferred_element_type=f32)
         + norm * wcr[...] + be1[...])
    m = x * jax.nn.sigmoid(x)
    x = jnp.dot(m, we2[...], preferred_element_type=f32) + be2[...]
    m = x * jax.nn.sigmoid(x)
    x = jnp.dot(m, wc1[...], preferred_element_type=f32) + bc1[...]
    c = x * jax.nn.sigmoid(x)
    c = jnp.dot(c, wc2[...], preferred_element_type=f32) + bc2[...]
    gate = jax.nn.sigmoid(jnp.dot(m, wi[...], preferred_element_type=f32) + bi[...])
    out_m[...] = m * gate
    bsz = cd.shape[0]
    out_c[...] = jnp.concatenate([cd * c, jnp.zeros((bsz, 1), f32)], axis=1)


def _pc_reduce_body(*refs):
    out = refs[-1]
    acc = refs[0][...]
    for r in refs[1:-1]:
        acc = acc + r[...]
    out[...] = jnp.sum(acc, axis=0)


def _node_body(nf, aggc3, wv1, bv1, wv2, bv2, wn1a, wn1b, bn1, wn2, bn2,
               *pms_and_out):
    pms = pms_and_out[:-1]
    out = pms_and_out[-1]
    f32 = jnp.float32
    coords = nf[:, 0:3]
    vels = nf[:, 3:6]
    h = nf[:, 6:6 + D]
    aggm = pms[0][0] + pms[0][1]
    for pm in pms[1:]:
        aggm = aggm + pm[0] + pm[1]
    aggc = aggc3[...]
    x = jnp.dot(h, wv1[...], preferred_element_type=f32) + bv1[...]
    v = x * jax.nn.sigmoid(x)
    v = jnp.dot(v, wv2[...], preferred_element_type=f32) + bv2[...]
    coords_new = coords + aggc + v * vels
    x = (jnp.dot(h, wn1a[...], preferred_element_type=f32)
         + jnp.dot(aggm, wn1b[...], preferred_element_type=f32) + bn1[...])
    hn = x * jax.nn.sigmoid(x)
    hn = jnp.dot(hn, wn2[...], preferred_element_type=f32) + bn2[...]
    out[...] = jnp.concatenate([coords_new, vels, h + hn], axis=1)


def kernel(node_features_input, edge_features, W_e1, b_e1, W_e2, b_e2,
           W_c1, b_c1, W_c2, b_c2, W_v1, b_v1, W_v2, b_v2, W_n1, b_n1,
           W_n2, b_n2, W_i, b_i, edge_indices):
    f32 = jnp.float32
    nf = node_features_input
    h_table = nf[:, 6:]
    ctab = nf[:, :3].reshape(-1)
    starts2d = edge_indices[0].reshape(NCHUNKS, CHUNK)
    ends2d = edge_indices[1].reshape(NCHUNKS, CHUNK)

    zeros_m = jnp.zeros((ROWS_PER_TILE, D), f32)
    zeros_c = jnp.zeros((3 * N,), f32)

    BE = 8000
    grid_e = ES // BE
    full = lambda i: (0, 0)
    edge_weight_specs = [
        pl.BlockSpec((D, H), full),
        pl.BlockSpec((D, H), full),
        pl.BlockSpec((1, H), full),
        pl.BlockSpec((DE, H), full),
        pl.BlockSpec((1, H), full),
        pl.BlockSpec((H, H), full),
        pl.BlockSpec((1, H), full),
        pl.BlockSpec((H, H), full),
        pl.BlockSpec((1, H), full),
        pl.BlockSpec((H, 1), full),
        pl.BlockSpec((1, 1), full),
        pl.BlockSpec((H, 1), full),
        pl.BlockSpec((1, 1), full),
    ]
    edge_weights = (
        W_e1[:D], W_e1[D:2 * D], W_e1[2 * D:2 * D + 1], W_e1[2 * D + 1:],
        b_e1.reshape(1, H), W_e2, b_e2.reshape(1, H), W_c1,
        b_c1.reshape(1, H), W_c2, b_c2.reshape(1, 1), W_i, b_i.reshape(1, 1))

    partial_ms = []
    partial_cs = []
    for k in range(S):
        st_k = lax.slice_in_dim(starts2d, k * NCH_S, (k + 1) * NCH_S)
        en_k = lax.slice_in_dim(ends2d, k * NCH_S, (k + 1) * NCH_S)
        gs, ge, cdn = _gather_s(h_table, ctab, st_k, en_k)

        row_block = lambda i: (i, 0)
        ef_block = lambda i, k=k: (k * grid_e + i, 0)
        mvals, cterm = pl.pallas_call(
            _edge_body,
            grid=(grid_e,),
            in_specs=[
                pl.BlockSpec((BE, D), row_block),
                pl.BlockSpec((BE, D), row_block),
                pl.BlockSpec((BE, 4), row_block),
                pl.BlockSpec((BE, DE), ef_block),
            ] + edge_weight_specs,
            out_specs=[
                pl.BlockSpec((BE, D), row_block),
                pl.BlockSpec((BE, 4), row_block),
            ],
            out_shape=[
                jax.ShapeDtypeStruct((ES, D), f32),
                jax.ShapeDtypeStruct((ES, 4), f32),
            ],
        )(gs, ge, cdn, edge_features, *edge_weights)

        partial_cs.append(_scatter_c_s(cterm, st_k, zeros_c))
        partial_ms.append(_scatter_m_s(mvals, st_k, zeros_m))

    agg_c = pl.pallas_call(
        _pc_reduce_body,
        grid=(1,),
        in_specs=[pl.BlockSpec((NW, 3 * N), lambda i: (0, 0))] * S,
        out_specs=pl.BlockSpec((3 * N,), lambda i: (0,)),
        out_shape=jax.ShapeDtypeStruct((3 * N,), f32),
    )(*partial_cs).reshape(N, 3)

    BN = 1000
    grid_n = N // BN
    out = pl.pallas_call(
        _node_body,
        grid=(grid_n,),
        in_specs=[
            pl.BlockSpec((BN, 6 + D), lambda i: (i, 0)),
            pl.BlockSpec((BN, 3), lambda i: (i, 0)),
            pl.BlockSpec((D, H), lambda i: (0, 0)),
            pl.BlockSpec((1, H), lambda i: (0, 0)),
            pl.BlockSpec((H, 1), lambda i: (0, 0)),
            pl.BlockSpec((1, 1), lambda i: (0, 0)),
            pl.BlockSpec((D, H), lambda i: (0, 0)),
            pl.BlockSpec((H, H), lambda i: (0, 0)),
            pl.BlockSpec((1, H), lambda i: (0, 0)),
            pl.BlockSpec((H, D), lambda i: (0, 0)),
            pl.BlockSpec((1, D), lambda i: (0, 0)),
        ] + [pl.BlockSpec((NC, BN, D), lambda i: (0, i, 0))] * S,
        out_specs=pl.BlockSpec((BN, 6 + D), lambda i: (i, 0)),
        out_shape=jax.ShapeDtypeStruct((N, 6 + D), f32),
    )(nf, agg_c, W_v1, b_v1.reshape(1, H), W_v2, b_v2.reshape(1, 1),
      W_n1[:D], W_n1[D:], b_n1.reshape(1, H), W_n2, b_n2.reshape(1, D),
      *partial_ms)

    return out


# speedup vs baseline: 1.0967x; 1.0098x over previous
import functools

import jax
import jax.numpy as jnp
from jax import lax
from jax.experimental import pallas as pl
from jax.experimental.pallas import tpu as pltpu
from jax.experimental.pallas import tpu_sc as plsc

N = 10000
E = 320000
D = 128
H = 128
DE = 16

CHUNK = 128
NCHUNKS = E // CHUNK
NC = 2
NS = 16
NW = NC * NS
L = 16
NP = 10240
ROWS_PER_TILE = NP // NS

S = 2
NCH_LIST = (1500, 1000)
CHUNK_OFF = (0, 1500)

_mesh = plsc.VectorSubcoreMesh(core_axis_name="c", subcore_axis_name="s")
_sc_params = pltpu.CompilerParams(needs_layout_passes=False)


def _make_gather(nch):
    n_edges = nch * CHUNK
    iters = (nch + NW - 1) // NW
    iters += iters % 2
    half = iters // 2

    @functools.partial(
        pl.kernel,
        out_type=(
            jax.ShapeDtypeStruct((n_edges, D), jnp.float32),
            jax.ShapeDtypeStruct((n_edges, D), jnp.float32),
            jax.ShapeDtypeStruct((n_edges, 4), jnp.float32),
        ),
        mesh=_mesh,
        compiler_params=_sc_params,
        scratch_types=[
            pltpu.VMEM((3 * N,), jnp.float32),
            pltpu.VMEM((CHUNK,), jnp.int32),
            pltpu.VMEM((CHUNK,), jnp.int32),
            pltpu.VMEM((CHUNK,), jnp.int32),
            pltpu.VMEM((CHUNK,), jnp.int32),
            pltpu.VMEM((CHUNK, D), jnp.float32),
            pltpu.VMEM((CHUNK, D), jnp.float32),
            pltpu.VMEM((CHUNK, D), jnp.float32),
            pltpu.VMEM((CHUNK, D), jnp.float32),
            pltpu.VMEM((CHUNK, 4), jnp.float32),
            pltpu.VMEM((CHUNK, 4), jnp.float32),
            pltpu.SemaphoreType.DMA,
            pltpu.SemaphoreType.DMA,
            pltpu.SemaphoreType.DMA,
            pltpu.SemaphoreType.DMA,
            pltpu.SemaphoreType.DMA,
            pltpu.SemaphoreType.DMA,
        ],
    )
    def gather(table_hbm, ctab_hbm, starts_hbm, ends_hbm,
               gs_hbm, ge_hbm, cdn_hbm,
               ctab_v, idx_s0, idx_e0, idx_s1, idx_e1,
               rows_s0, rows_e0, rows_s1, rows_e1, cdn0, cdn1,
               sem_i0, sem_i1, sem_g0, sem_g1, sem_w0, sem_w1):
        wid = lax.axis_index("s") * NC + lax.axis_index("c")
        pltpu.sync_copy(ctab_hbm, ctab_v)

        bufs = (
            (idx_s0, idx_e0, rows_s0, rows_e0, cdn0, sem_i0, sem_g0, sem_w0),
            (idx_s1, idx_e1, rows_s1, rows_e1, cdn1, sem_i1, sem_g1, sem_w1),
        )

        def start_idx(p, chunk):
            idx_s, idx_e, _, _, _, sem_i, _, _ = bufs[p]
            pltpu.async_copy(starts_hbm.at[chunk], idx_s, sem_i)
            pltpu.async_copy(ends_hbm.at[chunk], idx_e, sem_i)

        def wait_idx(p, chunk):
            idx_s, idx_e, _, _, _, sem_i, _, _ = bufs[p]
            pltpu.make_async_copy(starts_hbm.at[chunk], idx_s, sem_i).wait()
            pltpu.make_async_copy(ends_hbm.at[chunk], idx_e, sem_i).wait()

        def start_gather(p):
            idx_s, idx_e, rows_s, rows_e, _, _, sem_g, _ = bufs[p]
            pltpu.async_copy(table_hbm.at[idx_s], rows_s, sem_g)
            pltpu.async_copy(table_hbm.at[idx_e], rows_e, sem_g)

        def wait_gather(p):
            idx_s, idx_e, rows_s, rows_e, _, _, sem_g, _ = bufs[p]
            pltpu.make_async_copy(table_hbm.at[idx_s], rows_s, sem_g).wait()
            pltpu.make_async_copy(table_hbm.at[idx_e], rows_e, sem_g).wait()

        def start_write(p, chunk):
            _, _, rows_s, rows_e, cdn_v, _, _, sem_w = bufs[p]
            sl = pl.ds(chunk * CHUNK, CHUNK)
            pltpu.async_copy(rows_s, gs_hbm.at[sl], sem_w)
            pltpu.async_copy(rows_e, ge_hbm.at[sl], sem_w)
            pltpu.async_copy(cdn_v, cdn_hbm.at[sl], sem_w)

        def wait_write(p, chunk):
            _, _, rows_s, rows_e, cdn_v, _, _, sem_w = bufs[p]
            sl = pl.ds(chunk * CHUNK, CHUNK)
            pltpu.make_async_copy(rows_s, gs_hbm.at[sl], sem_w).wait()
            pltpu.make_async_copy(rows_e, ge_hbm.at[sl], sem_w).wait()
            pltpu.make_async_copy(cdn_v, cdn_hbm.at[sl], sem_w).wait()

        def coords(p):
            idx_s, idx_e, _, _, cdn_v, _, _, _ = bufs[p]
            for j in range(CHUNK // L):
                lanes = lax.iota(jnp.int32, L) + j * L
                i_s = idx_s[pl.ds(j * L, L)] * 3
                i_e = idx_e[pl.ds(j * L, L)] * 3
                cdx = plsc.load_gather(ctab_v, [i_s]) - plsc.load_gather(ctab_v, [i_e])
                cdy = plsc.load_gather(ctab_v, [i_s + 1]) - plsc.load_gather(ctab_v, [i_e + 1])
                cdz = plsc.load_gather(ctab_v, [i_s + 2]) - plsc.load_gather(ctab_v, [i_e + 2])
                n2 = cdx * cdx + cdy * cdy + cdz * cdz + 1e-12
                c0 = jnp.zeros((L,), jnp.int32)
                plsc.store_scatter(cdn_v, [lanes, c0], cdx)
                plsc.store_scatter(cdn_v, [lanes, c0 + 1], cdy)
                plsc.store_scatter(cdn_v, [lanes, c0 + 2], cdz)
                plsc.store_scatter(cdn_v, [lanes, c0 + 3], n2)

        start_idx(0, wid)

        def body(k, carry):
            for p in range(2):
                i = 2 * k + p
                c_i = wid + i * NW
                c_n = c_i + NW
                c_o = c_i - 2 * NW

                @pl.when(c_i < nch)
                def _():
                    wait_idx(p, c_i)

                @pl.when(jnp.logical_and(i >= 2, c_o < nch))
                def _():
                    wait_write(p, c_o)

                @pl.when(c_i < nch)
                def _():
                    start_gather(p)

                @pl.when(c_n < nch)
                def _():
                    start_idx(1 - p, c_n)

                @pl.when(c_i < nch)
                def _():
                    coords(p)
                    wait_gather(p)
                    start_write(p, c_i)

            return carry

        lax.fori_loop(0, half, body, 0)

        for p in range(2):
            i = iters - 2 + p
            c_i = wid + i * NW

            @pl.when(c_i < nch)
            def _():
                wait_write(p, c_i)

    return gather


def _make_scatter_m(nch):
    chunks_per_core = nch // NC
    iters = (chunks_per_core + NS - 1) // NS
    iters += iters % 2

    @functools.partial(
        pl.kernel,
        out_type=jax.ShapeDtypeStruct((NC, NP, D), jnp.float32),
        mesh=_mesh,
        compiler_params=_sc_params,
        scratch_types=[
            pltpu.VMEM_SHARED((NP, D), jnp.float32),
            pltpu.VMEM((CHUNK,), jnp.int32),
            pltpu.VMEM((CHUNK,), jnp.int32),
            pltpu.VMEM((CHUNK, D), jnp.float32),
            pltpu.VMEM((CHUNK, D), jnp.float32),
            pltpu.SemaphoreType.DMA,
            pltpu.SemaphoreType.DMA,
            pltpu.SemaphoreType.DMA,
            pltpu.SemaphoreType.DMA,
        ],
    )
    def scatter_m(mvals_hbm, starts_hbm, zeros_hbm, out_m_hbm, acc_m,
                  idx0, idx1, rows0, rows1, sem_l0, sem_l1, sem_a0, sem_a1):
        cid = lax.axis_index("c")
        sid = lax.axis_index("s")
        pltpu.sync_copy(zeros_hbm,
                        acc_m.at[pl.ds(sid * ROWS_PER_TILE, ROWS_PER_TILE)])
        plsc.subcore_barrier()

        bufs = ((idx0, rows0, sem_l0, sem_a0), (idx1, rows1, sem_l1, sem_a1))

        def chunk_of(local):
            return cid * chunks_per_core + local

        def start_loads(p, local):
            idx_v, rows_v, sem_l, _ = bufs[p]
            c = chunk_of(local)
            pltpu.async_copy(starts_hbm.at[c], idx_v, sem_l)
            pltpu.async_copy(mvals_hbm.at[pl.ds(c * CHUNK, CHUNK)], rows_v, sem_l)

        def wait_loads(p, local):
            idx_v, rows_v, sem_l, _ = bufs[p]
            c = chunk_of(local)
            pltpu.make_async_copy(starts_hbm.at[c], idx_v, sem_l).wait()
            pltpu.make_async_copy(mvals_hbm.at[pl.ds(c * CHUNK, CHUNK)], rows_v, sem_l).wait()

        def start_add(p):
            idx_v, rows_v, _, sem_a = bufs[p]
            pltpu.async_copy(rows_v, acc_m.at[idx_v], sem_a, add=True)

        def wait_add(p):
            idx_v, rows_v, _, sem_a = bufs[p]
            pltpu.make_async_copy(rows_v, acc_m.at[idx_v], sem_a).wait()

        start_loads(0, sid)

        def body(k, carry):
            for p in range(2):
                i = 2 * k + p
                l_i = sid + i * NS
                l_n = l_i + NS
                l_o = l_i - 2 * NS

                @pl.when(jnp.logical_and(i >= 2, l_o < chunks_per_core))
                def _():
                    wait_add(p)

                @pl.when(l_i < chunks_per_core)
                def _():
                    wait_loads(p, l_i)
                    start_add(p)

                @pl.when(l_n < chunks_per_core)
                def _():
                    start_loads(1 - p, l_n)

            return carry

        lax.fori_loop(0, iters // 2, body, 0)

        for p in range(2):
            i = iters - 2 + p
            l_i = sid + i * NS

            @pl.when(l_i < chunks_per_core)
            def _():
                wait_add(p)

        plsc.subcore_barrier()
        pltpu.sync_copy(
            acc_m.at[pl.ds(sid * ROWS_PER_TILE, ROWS_PER_TILE)],
            out_m_hbm.at[cid, pl.ds(sid * ROWS_PER_TILE, ROWS_PER_TILE)],
        )

    return scatter_m


def _make_scatter_c(nch):
    iters = (nch + NW - 1) // NW
    iters += iters % 2

    @functools.partial(
        pl.kernel,
        out_type=jax.ShapeDtypeStruct((NW, 3 * N), jnp.float32),
        mesh=_mesh,
        compiler_params=_sc_params,
        scratch_types=[
            pltpu.VMEM((3 * N,), jnp.float32),
            pltpu.VMEM((CHUNK,), jnp.int32),
            pltpu.VMEM((CHUNK,), jnp.int32),
            pltpu.VMEM((CHUNK, 4), jnp.float32),
            pltpu.VMEM((CHUNK, 4), jnp.float32),
            pltpu.SemaphoreType.DMA,
            pltpu.SemaphoreType.DMA,
        ],
    )
    def scatter_c(cterm_hbm, starts_hbm, zeros_c_hbm, out_c_hbm, acc_c,
                  idx0, idx1, rows0, rows1, sem_l0, sem_l1):
        cid = lax.axis_index("c")
        sid = lax.axis_index("s")
        wid = sid * NC + cid
        pltpu.sync_copy(zeros_c_hbm, acc_c)

        bufs = ((idx0, rows0, sem_l0), (idx1, rows1, sem_l1))

        def start_loads(p, chunk):
            idx_v, rows_c, sem_l = bufs[p]
            pltpu.async_copy(starts_hbm.at[chunk], idx_v, sem_l)
            pltpu.async_copy(cterm_hbm.at[pl.ds(chunk * CHUNK, CHUNK)], rows_c, sem_l)

        def wait_loads(p, chunk):
            idx_v, rows_c, sem_l = bufs[p]
            pltpu.make_async_copy(starts_hbm.at[chunk], idx_v, sem_l).wait()
            pltpu.make_async_copy(cterm_hbm.at[pl.ds(chunk * CHUNK, CHUNK)], rows_c, sem_l).wait()

        def compute(p):
            idx_v, rows_c, _ = bufs[p]
            for j in range(CHUNK // L):
                lanes = lax.iota(jnp.int32, L) + j * L
                tgt = idx_v[pl.ds(j * L, L)] * 3
                c0 = jnp.zeros((L,), jnp.int32)
                cx = plsc.load_gather(rows_c, [lanes, c0])
                cy = plsc.load_gather(rows_c, [lanes, c0 + 1])
                cz = plsc.load_gather(rows_c, [lanes, c0 + 2])
                plsc.addupdate_scatter(acc_c, [tgt], cx)
                plsc.addupdate_scatter(acc_c, [tgt + 1], cy)
                plsc.addupdate_scatter(acc_c, [tgt + 2], cz)

        start_loads(0, wid)

        def body(k, carry):
            for p in range(2):
                i = 2 * k + p
                c_i = wid + i * NW
                c_n = c_i + NW

                @pl.when(c_n < nch)
                def _():
                    start_loads(1 - p, c_n)

                @pl.when(c_i < nch)
                def _():
                    wait_loads(p, c_i)
                    compute(p)

            return carry

        lax.fori_loop(0, iters // 2, body, 0)
        pltpu.sync_copy(acc_c, out_c_hbm.at[wid])

    return scatter_c


_gathers = tuple(_make_gather(n) for n in NCH_LIST)
_scatter_ms = tuple(_make_scatter_m(n) for n in NCH_LIST)
_scatter_cs = tuple(_make_scatter_c(n) for n in NCH_LIST)


def _edge_body(gs, ge, cdn, ef, wa, wb, wcr, wd, be1, we2, be2, wc1, bc1,
               wc2, bc2, wi, bi, out_m, out_c):
    f32 = jnp.float32
    cd = cdn[:, 0:3]
    norm = jnp.sqrt(cdn[:, 3:4])
    x = (jnp.dot(gs[...], wa[...], preferred_element_type=f32)
         + jnp.dot(ge[...], wb[...], preferred_element_type=f32)
         + jnp.dot(ef[...], wd[...], preferred_element_type=f32)
         + norm * wcr[...] + be1[...])
    m = x * jax.nn.sigmoid(x)
    x = jnp.dot(m, we2[...], preferred_element_type=f32) + be2[...]
    m = x * jax.nn.sigmoid(x)
    x = jnp.dot(m, wc1[...], preferred_element_type=f32) + bc1[...]
    c = x * jax.nn.sigmoid(x)
    c = jnp.dot(c, wc2[...], preferred_element_type=f32) + bc2[...]
    gate = jax.nn.sigmoid(jnp.dot(m, wi[...], preferred_element_type=f32) + bi[...])
    out_m[...] = m * gate
    bsz = cd.shape[0]
    out_c[...] = jnp.concatenate([cd * c, jnp.zeros((bsz, 1), f32)], axis=1)


def _pc_reduce_body(*refs):
    out = refs[-1]
    acc = refs[0][...]
    for r in refs[1:-1]:
        acc = acc + r[...]
    out[...] = jnp.sum(acc, axis=0)


def _node_body(nf, aggc3, wv1, bv1, wv2, bv2, wn1a, wn1b, bn1, wn2, bn2,
               *pms_and_out):
    pms = pms_and_out[:-1]
    out = pms_and_out[-1]
    f32 = jnp.float32
    coords = nf[:, 0:3]
    vels = nf[:, 3:6]
    h = nf[:, 6:6 + D]
    aggm = pms[0][0] + pms[0][1]
    for pm in pms[1:]:
        aggm = aggm + pm[0] + pm[1]
    aggc = aggc3[...]
    x = jnp.dot(h, wv1[...], preferred_element_type=f32) + bv1[...]
    v = x * jax.nn.sigmoid(x)
    v = jnp.dot(v, wv2[...], preferred_element_type=f32) + bv2[...]
    coords_new = coords + aggc + v * vels
    x = (jnp.dot(h, wn1a[...], preferred_element_type=f32)
         + jnp.dot(aggm, wn1b[...], preferred_element_type=f32) + bn1[...])
    hn = x * jax.nn.sigmoid(x)
    hn = jnp.dot(hn, wn2[...], preferred_element_type=f32) + bn2[...]
    out[...] = jnp.concatenate([coords_new, vels, h + hn], axis=1)


def kernel(node_features_input, edge_features, W_e1, b_e1, W_e2, b_e2,
           W_c1, b_c1, W_c2, b_c2, W_v1, b_v1, W_v2, b_v2, W_n1, b_n1,
           W_n2, b_n2, W_i, b_i, edge_indices):
    f32 = jnp.float32
    nf = node_features_input
    h_table = nf[:, 6:]
    ctab = nf[:, :3].reshape(-1)
    starts2d = edge_indices[0].reshape(NCHUNKS, CHUNK)
    ends2d = edge_indices[1].reshape(NCHUNKS, CHUNK)

    zeros_m = jnp.zeros((ROWS_PER_TILE, D), f32)
    zeros_c = jnp.zeros((3 * N,), f32)

    BE = 8000
    full = lambda i: (0, 0)
    edge_weight_specs = [
        pl.BlockSpec((D, H), full),
        pl.BlockSpec((D, H), full),
        pl.BlockSpec((1, H), full),
        pl.BlockSpec((DE, H), full),
        pl.BlockSpec((1, H), full),
        pl.BlockSpec((H, H), full),
        pl.BlockSpec((1, H), full),
        pl.BlockSpec((H, H), full),
        pl.BlockSpec((1, H), full),
        pl.BlockSpec((H, 1), full),
        pl.BlockSpec((1, 1), full),
        pl.BlockSpec((H, 1), full),
        pl.BlockSpec((1, 1), full),
    ]
    edge_weights = (
        W_e1[:D], W_e1[D:2 * D], W_e1[2 * D:2 * D + 1], W_e1[2 * D + 1:],
        b_e1.reshape(1, H), W_e2, b_e2.reshape(1, H), W_c1,
        b_c1.reshape(1, H), W_c2, b_c2.reshape(1, 1), W_i, b_i.reshape(1, 1))

    partial_ms = []
    partial_cs = []
    for k in range(S):
        nch_k = NCH_LIST[k]
        off_k = CHUNK_OFF[k]
        ES = nch_k * CHUNK
        grid_e = ES // BE
        blk_off = (off_k * CHUNK) // BE
        st_k = lax.slice_in_dim(starts2d, off_k, off_k + nch_k)
        en_k = lax.slice_in_dim(ends2d, off_k, off_k + nch_k)
        gs, ge, cdn = _gathers[k](h_table, ctab, st_k, en_k)

        row_block = lambda i: (i, 0)
        ef_block = lambda i, o=blk_off: (o + i, 0)
        mvals, cterm = pl.pallas_call(
            _edge_body,
            grid=(grid_e,),
            in_specs=[
                pl.BlockSpec((BE, D), row_block),
                pl.BlockSpec((BE, D), row_block),
                pl.BlockSpec((BE, 4), row_block),
                pl.BlockSpec((BE, DE), ef_block),
            ] + edge_weight_specs,
            out_specs=[
                pl.BlockSpec((BE, D), row_block),
                pl.BlockSpec((BE, 4), row_block),
            ],
            out_shape=[
                jax.ShapeDtypeStruct((ES, D), f32),
                jax.ShapeDtypeStruct((ES, 4), f32),
            ],
        )(gs, ge, cdn, edge_features, *edge_weights)

        partial_cs.append(_scatter_cs[k](cterm, st_k, zeros_c))
        partial_ms.append(_scatter_ms[k](mvals, st_k, zeros_m))

    agg_c = pl.pallas_call(
        _pc_reduce_body,
        grid=(1,),
        in_specs=[pl.BlockSpec((NW, 3 * N), lambda i: (0, 0))] * S,
        out_specs=pl.BlockSpec((3 * N,), lambda i: (0,)),
        out_shape=jax.ShapeDtypeStruct((3 * N,), f32),
    )(*partial_cs).reshape(N, 3)

    BN = 1000
    grid_n = N // BN
    out = pl.pallas_call(
        _node_body,
        grid=(grid_n,),
        in_specs=[
            pl.BlockSpec((BN, 6 + D), lambda i: (i, 0)),
            pl.BlockSpec((BN, 3), lambda i: (i, 0)),
            pl.BlockSpec((D, H), lambda i: (0, 0)),
            pl.BlockSpec((1, H), lambda i: (0, 0)),
            pl.BlockSpec((H, 1), lambda i: (0, 0)),
            pl.BlockSpec((1, 1), lambda i: (0, 0)),
            pl.BlockSpec((D, H), lambda i: (0, 0)),
            pl.BlockSpec((H, H), lambda i: (0, 0)),
            pl.BlockSpec((1, H), lambda i: (0, 0)),
            pl.BlockSpec((H, D), lambda i: (0, 0)),
            pl.BlockSpec((1, D), lambda i: (0, 0)),
        ] + [pl.BlockSpec((NC, BN, D), lambda i: (0, i, 0))] * S,
        out_specs=pl.BlockSpec((BN, 6 + D), lambda i: (i, 0)),
        out_shape=jax.ShapeDtypeStruct((N, 6 + D), f32),
    )(nf, agg_c, W_v1, b_v1.reshape(1, H), W_v2, b_v2.reshape(1, 1),
      W_n1[:D], W_n1[D:], b_n1.reshape(1, H), W_n2, b_n2.reshape(1, D),
      *partial_ms)

    return out
